# baseline (speedup 1.0000x reference)
import functools

import jax
import jax.numpy as jnp
from jax import lax
from jax.experimental import pallas as pl
from jax.experimental.pallas import tpu as pltpu
from jax.experimental.pallas import tpu_sc as plsc

_EPS = 1e-05
_W = 8192
_C = 32
_MOMENTUM = 0.99

_F = 16384
_M_TILE = 1024
_N_TILES = _F // _M_TILE

_NC = 2
_NS = 16
_NW = _NC * _NS
_L = 16
_SC_IT = _F // _L
_ZB_IT = _W // _L


def _argmax_body(f_ref, d_ref, idx_ref, dn_ref):
    d = d_ref[...]
    nrm = jnp.sqrt(jnp.sum(d * d, axis=1, keepdims=True))
    dn_ref[...] = d / jnp.maximum(nrm, _EPS)

    def body(i, carry):
        f = f_ref[pl.ds(i * _M_TILE, _M_TILE), :].astype(jnp.bfloat16)
        sim = lax.dot_general(
            f, dn_ref[...].astype(jnp.bfloat16), (((1,), (1,)), ((), ())),
            preferred_element_type=jnp.float32)
        mx = jnp.max(sim, axis=1, keepdims=True)
        j = lax.broadcasted_iota(jnp.int32, sim.shape, 1)
        idx = jnp.min(jnp.where(sim >= mx, j, jnp.int32(2**30)), axis=1)
        idx_ref[i, :] = idx
        return carry

    lax.fori_loop(0, _N_TILES, body, 0)


def _argmax_call(feature, dictionary):
    return pl.pallas_call(
        _argmax_body,
        out_shape=jax.ShapeDtypeStruct((_N_TILES, _M_TILE), jnp.int32),
        scratch_shapes=[pltpu.VMEM((_W, _C), jnp.float32)],
    )(feature, dictionary)


def _scatter_body(valsT_hbm, idx_hbm, o_hbm, val_v, idx_v, acc):
    c = lax.axis_index("c")
    s = lax.axis_index("s")
    t = c * _NS + s
    pltpu.sync_copy(valsT_hbm.at[pl.ds(t * _F, _F)], val_v)
    pltpu.sync_copy(idx_hbm, idx_v)

    def zero(k, carry):
        acc[pl.ds(k * _L, _L)] = jnp.zeros((_L,), jnp.float32)
        return carry
    lax.fori_loop(0, _ZB_IT, zero, 0)

    def body(k, carry):
        ii = idx_v[pl.ds(k * _L, _L)]
        xx = val_v[pl.ds(k * _L, _L)]
        plsc.addupdate_scatter(acc, [ii], xx)
        return carry
    lax.fori_loop(0, _SC_IT, body, 0)
    pltpu.sync_copy(acc, o_hbm.at[pl.ds(t * _W, _W)])

    @pl.when(t == _NW - 1)
    def _():
        def zero2(k, carry):
            acc[pl.ds(k * _L, _L)] = jnp.zeros((_L,), jnp.float32)
            return carry
        lax.fori_loop(0, _ZB_IT, zero2, 0)
        ones = jnp.ones((_L,), jnp.float32)

        def body2(k, carry):
            ii = idx_v[pl.ds(k * _L, _L)]
            plsc.addupdate_scatter(acc, [ii], ones)
            return carry
        lax.fori_loop(0, _SC_IT, body2, 0)
        pltpu.sync_copy(acc, o_hbm.at[pl.ds(_NW * _W, _W)])


@functools.lru_cache(maxsize=1)
def _make_scatter_call():
    return pl.kernel(
        _scatter_body,
        out_type=jax.ShapeDtypeStruct(((_NW + 1) * _W,), jnp.float32),
        mesh=plsc.VectorSubcoreMesh(core_axis_name="c", subcore_axis_name="s",
                                    num_cores=_NC, num_subcores=_NS),
        compiler_params=pltpu.CompilerParams(needs_layout_passes=False),
        scratch_types=[
            pltpu.VMEM((_F,), jnp.float32),
            pltpu.VMEM((_F,), jnp.int32),
            pltpu.VMEM((_W,), jnp.float32),
        ],
    )


def _ema_body(p_ref, d_ref, ds_ref, dn_ref, o_ref):
    p = p_ref[...]
    seg = p[:, :_C]
    cnt = p[:, _C:_C + 1]
    used = cnt > 0.0
    dsum = ds_ref[...]
    dnum = dn_ref[...]
    new_sum = jnp.where(used, _MOMENTUM * dsum + (1.0 - _MOMENTUM) * seg, dsum)
    new_num = jnp.where(used, _MOMENTUM * dnum + (1.0 - _MOMENTUM) * cnt, dnum)
    o_ref[...] = jnp.where(used, new_sum / new_num, d_ref[...])


def _ema_call(seg, dictionary, dictionary_sum, dictionary_num):
    return pl.pallas_call(
        _ema_body,
        out_shape=jax.ShapeDtypeStruct((_W, _C), jnp.float32),
    )(seg, dictionary, dictionary_sum, dictionary_num)


def kernel(feature, dictionary, dictionary_sum, dictionary_num):
    feature = feature.reshape(-1, _C).astype(jnp.float32)
    idx = _argmax_call(feature, dictionary)
    featT = feature.T.reshape(-1)
    flat = _make_scatter_call()(featT, idx.reshape(-1))
    seg = flat.reshape(_NW + 1, _W).T
    return _ema_call(seg, dictionary, dictionary_sum,
                     dictionary_num.reshape(_W, 1))


# speedup vs baseline: 1.3683x; 1.3683x over previous
import functools

import jax
import jax.numpy as jnp
from jax import lax
from jax.experimental import pallas as pl
from jax.experimental.pallas import tpu as pltpu
from jax.experimental.pallas import tpu_sc as plsc

_EPS = 1e-05
_W = 8192
_C = 32
_MOMENTUM = 0.99

_F = 16384
_M_TILE = 1024
_N_TILES = _F // _M_TILE

_NC = 2
_NS = 16
_NW = _NC * _NS
_L = 16
_SC_IT = _F // _L
_ZB_IT = _W // _L


def _argmax_body(f_ref, d_ref, idx_ref, dn_ref):
    d = d_ref[...]
    nrm = jnp.sqrt(jnp.sum(d * d, axis=1, keepdims=True))
    dn_ref[...] = d / jnp.maximum(nrm, _EPS)

    def body(i, carry):
        f = f_ref[pl.ds(i * _M_TILE, _M_TILE), :].astype(jnp.bfloat16)
        sim = lax.dot_general(
            f, dn_ref[...].astype(jnp.bfloat16), (((1,), (1,)), ((), ())),
            preferred_element_type=jnp.float32)
        j = lax.broadcasted_iota(jnp.int32, sim.shape, 1)
        enc = (lax.bitcast_convert_type(sim, jnp.int32)
               & jnp.int32(~0x1FFF)) | (jnp.int32(8191) - j)
        best = jnp.max(lax.bitcast_convert_type(enc, jnp.float32), axis=1)
        idx = jnp.int32(8191) - (
            lax.bitcast_convert_type(best, jnp.int32) & jnp.int32(0x1FFF))
        idx_ref[i, :] = idx
        return carry

    lax.fori_loop(0, _N_TILES, body, 0)


def _argmax_call(feature, dictionary):
    return pl.pallas_call(
        _argmax_body,
        out_shape=jax.ShapeDtypeStruct((_N_TILES, _M_TILE), jnp.int32),
        scratch_shapes=[pltpu.VMEM((_W, _C), jnp.float32)],
    )(feature, dictionary)


def _scatter_body(valsT_hbm, idx_hbm, o_hbm, val_v, idx_v, acc):
    c = lax.axis_index("c")
    s = lax.axis_index("s")
    t = c * _NS + s
    pltpu.sync_copy(valsT_hbm.at[pl.ds(t * _F, _F)], val_v)
    pltpu.sync_copy(idx_hbm, idx_v)

    def zero(k, carry):
        acc[pl.ds(k * _L, _L)] = jnp.zeros((_L,), jnp.float32)
        return carry
    lax.fori_loop(0, _ZB_IT, zero, 0)

    def body(k, carry):
        ii = idx_v[pl.ds(k * _L, _L)]
        xx = val_v[pl.ds(k * _L, _L)]
        plsc.addupdate_scatter(acc, [ii], xx)
        return carry
    lax.fori_loop(0, _SC_IT, body, 0)
    pltpu.sync_copy(acc, o_hbm.at[pl.ds(t * _W, _W)])

    @pl.when(t == _NW - 1)
    def _():
        def zero2(k, carry):
            acc[pl.ds(k * _L, _L)] = jnp.zeros((_L,), jnp.float32)
            return carry
        lax.fori_loop(0, _ZB_IT, zero2, 0)
        ones = jnp.ones((_L,), jnp.float32)

        def body2(k, carry):
            ii = idx_v[pl.ds(k * _L, _L)]
            plsc.addupdate_scatter(acc, [ii], ones)
            return carry
        lax.fori_loop(0, _SC_IT, body2, 0)
        pltpu.sync_copy(acc, o_hbm.at[pl.ds(_NW * _W, _W)])


@functools.lru_cache(maxsize=1)
def _make_scatter_call():
    return pl.kernel(
        _scatter_body,
        out_type=jax.ShapeDtypeStruct(((_NW + 1) * _W,), jnp.float32),
        mesh=plsc.VectorSubcoreMesh(core_axis_name="c", subcore_axis_name="s",
                                    num_cores=_NC, num_subcores=_NS),
        compiler_params=pltpu.CompilerParams(needs_layout_passes=False),
        scratch_types=[
            pltpu.VMEM((_F,), jnp.float32),
            pltpu.VMEM((_F,), jnp.int32),
            pltpu.VMEM((_W,), jnp.float32),
        ],
    )


def _ema_body(p_ref, d_ref, ds_ref, dn_ref, o_ref):
    p = p_ref[...]
    seg = p[:, :_C]
    cnt = p[:, _C:_C + 1]
    used = cnt > 0.0
    dsum = ds_ref[...]
    dnum = dn_ref[...]
    new_sum = jnp.where(used, _MOMENTUM * dsum + (1.0 - _MOMENTUM) * seg, dsum)
    new_num = jnp.where(used, _MOMENTUM * dnum + (1.0 - _MOMENTUM) * cnt, dnum)
    o_ref[...] = jnp.where(used, new_sum / new_num, d_ref[...])


def _ema_call(seg, dictionary, dictionary_sum, dictionary_num):
    return pl.pallas_call(
        _ema_body,
        out_shape=jax.ShapeDtypeStruct((_W, _C), jnp.float32),
    )(seg, dictionary, dictionary_sum, dictionary_num)


def kernel(feature, dictionary, dictionary_sum, dictionary_num):
    feature = feature.reshape(-1, _C).astype(jnp.float32)
    idx = _argmax_call(feature, dictionary)
    featT = feature.T.reshape(-1)
    flat = _make_scatter_call()(featT, idx.reshape(-1))
    seg = flat.reshape(_NW + 1, _W).T
    return _ema_call(seg, dictionary, dictionary_sum,
                     dictionary_num.reshape(_W, 1))


# speedup vs baseline: 1.4067x; 1.0281x over previous
import functools

import jax
import jax.numpy as jnp
from jax import lax
from jax.experimental import pallas as pl
from jax.experimental.pallas import tpu as pltpu
from jax.experimental.pallas import tpu_sc as plsc

_EPS = 1e-05
_W = 8192
_C = 32
_MOMENTUM = 0.99

_F = 16384
_M_TILE = 1024
_N_TILES = _F // _M_TILE

_NC = 2
_NS = 16
_NW = _NC * _NS
_L = 16
_SC_IT = _F // _L
_ZB_IT = _W // _L
_CNT_TILES = 4


def _argmax_body(f_ref, d_ref, idx_ref, dn_ref):
    d = d_ref[...]
    nrm = jnp.sqrt(jnp.sum(d * d, axis=1, keepdims=True))
    dn_ref[...] = d / jnp.maximum(nrm, _EPS)

    def body(i, carry):
        f = f_ref[pl.ds(i * _M_TILE, _M_TILE), :].astype(jnp.bfloat16)
        sim = lax.dot_general(
            f, dn_ref[...].astype(jnp.bfloat16), (((1,), (1,)), ((), ())),
            preferred_element_type=jnp.float32)
        j = lax.broadcasted_iota(jnp.int32, sim.shape, 1)
        enc = (lax.bitcast_convert_type(sim, jnp.int32)
               & jnp.int32(~0x1FFF)) | j
        best = jnp.max(lax.bitcast_convert_type(enc, jnp.float32), axis=1)
        idx = lax.bitcast_convert_type(best, jnp.int32) & jnp.int32(0x1FFF)
        idx_ref[i, :] = idx
        return carry

    lax.fori_loop(0, _N_TILES, body, 0)


def _argmax_call(feature, dictionary):
    return pl.pallas_call(
        _argmax_body,
        out_shape=jax.ShapeDtypeStruct((_N_TILES, _M_TILE), jnp.int32),
        scratch_shapes=[pltpu.VMEM((_W, _C), jnp.float32)],
    )(feature, dictionary)


def _scatter_body(valsT_hbm, idx_hbm, o_hbm, val_v, idx_v, acc):
    c = lax.axis_index("c")
    s = lax.axis_index("s")
    t = c * _NS + s
    pltpu.sync_copy(valsT_hbm.at[pl.ds(t * _F, _F)], val_v)
    pltpu.sync_copy(idx_hbm, idx_v)

    def zero(k, carry):
        acc[pl.ds(k * _L, _L)] = jnp.zeros((_L,), jnp.float32)
        return carry
    lax.fori_loop(0, _ZB_IT, zero, 0)

    def body(k, carry):
        ii = idx_v[pl.ds(k * _L, _L)]
        xx = val_v[pl.ds(k * _L, _L)]
        plsc.addupdate_scatter(acc, [ii], xx)
        return carry
    lax.fori_loop(0, _SC_IT, body, 0)
    pltpu.sync_copy(acc, o_hbm.at[pl.ds(t * _W, _W)])

    @pl.when(t >= _NW - _CNT_TILES)
    def _():
        q = t - (_NW - _CNT_TILES)

        def zero2(k, carry):
            acc[pl.ds(k * _L, _L)] = jnp.zeros((_L,), jnp.float32)
            return carry
        lax.fori_loop(0, _ZB_IT, zero2, 0)
        ones = jnp.ones((_L,), jnp.float32)
        base = q * (_SC_IT // _CNT_TILES)

        def body2(k, carry):
            ii = idx_v[pl.ds((base + k) * _L, _L)]
            plsc.addupdate_scatter(acc, [ii], ones)
            return carry
        lax.fori_loop(0, _SC_IT // _CNT_TILES, body2, 0)
        pltpu.sync_copy(acc, o_hbm.at[pl.ds((_NW + q) * _W, _W)])


@functools.lru_cache(maxsize=1)
def _make_scatter_call():
    return pl.kernel(
        _scatter_body,
        out_type=jax.ShapeDtypeStruct(((_NW + _CNT_TILES) * _W,), jnp.float32),
        mesh=plsc.VectorSubcoreMesh(core_axis_name="c", subcore_axis_name="s",
                                    num_cores=_NC, num_subcores=_NS),
        compiler_params=pltpu.CompilerParams(needs_layout_passes=False),
        scratch_types=[
            pltpu.VMEM((_F,), jnp.float32),
            pltpu.VMEM((_F,), jnp.int32),
            pltpu.VMEM((_W,), jnp.float32),
        ],
    )


def _ema_body(p_ref, d_ref, ds_ref, dn_ref, o_ref):
    p = p_ref[...]
    seg = p[:, :_C]
    cnt = jnp.sum(p[:, _C:], axis=1, keepdims=True)
    used = cnt > 0.0
    dsum = ds_ref[...]
    dnum = dn_ref[...]
    new_sum = jnp.where(used, _MOMENTUM * dsum + (1.0 - _MOMENTUM) * seg, dsum)
    new_num = jnp.where(used, _MOMENTUM * dnum + (1.0 - _MOMENTUM) * cnt, dnum)
    o_ref[...] = jnp.where(used, new_sum / new_num, d_ref[...])


def _ema_call(seg, dictionary, dictionary_sum, dictionary_num):
    return pl.pallas_call(
        _ema_body,
        out_shape=jax.ShapeDtypeStruct((_W, _C), jnp.float32),
    )(seg, dictionary, dictionary_sum, dictionary_num)


def kernel(feature, dictionary, dictionary_sum, dictionary_num):
    feature = feature.reshape(-1, _C).astype(jnp.float32)
    idx = _argmax_call(feature, dictionary)
    featT = feature.T.reshape(-1)
    flat = _make_scatter_call()(featT, idx.reshape(-1))
    seg = flat.reshape(_NW + _CNT_TILES, _W).T
    return _ema_call(seg, dictionary, dictionary_sum,
                     dictionary_num.reshape(_W, 1))
